# Initial kernel scaffold; baseline (speedup 1.0000x reference)
import jax
import jax.numpy as jnp
from jax.experimental import pallas as pl


def kernel(scores, bbox_deltas, im_info):
    raise NotImplementedError("write your pallas kernel here")


# speedup vs baseline: 15.0217x; 15.0217x over previous
import functools

import numpy as np
import jax
import jax.numpy as jnp
from jax import lax
from jax.experimental import pallas as pl

FEAT_STRIDE = 16.0
PRE_NMS_TOP_N = 6000
POST_NMS_TOP_N = 300
NMS_THRESH = 0.7
MIN_SIZE = 16.0

_A = 9
_H = 50
_W = 44
_NV = _A * _H * _W
_ROWS = 160
_LANES = 128
_NPAD = _ROWS * _LANES


def _base_anchors():
    base = np.array([0.0, 0.0, 15.0, 15.0])
    w = base[2] - base[0] + 1
    h = base[3] - base[1] + 1
    x_ctr = base[0] + 0.5 * (w - 1)
    y_ctr = base[1] + 0.5 * (h - 1)
    size = w * h
    out = []
    for r in (0.5, 1.0, 2.0):
        ws = np.round(np.sqrt(size / r))
        hs = np.round(ws * r)
        for s in (8.0, 16.0, 32.0):
            ws_s = ws * s
            hs_s = hs * s
            out.append([x_ctr - 0.5 * (ws_s - 1), y_ctr - 0.5 * (hs_s - 1),
                        x_ctr + 0.5 * (ws_s - 1), y_ctr + 0.5 * (hs_s - 1)])
    return np.array(out, dtype=np.float64)

_BA = _base_anchors()
_BA_W = (_BA[:, 2] - _BA[:, 0] + 1.0).astype(np.float32)
_BA_H = (_BA[:, 3] - _BA[:, 1] + 1.0).astype(np.float32)
_BA_CX = (_BA[:, 0] + 0.5 * (_BA_W - 1.0) + 0.5).astype(np.float32)
_BA_CY = (_BA[:, 1] + 0.5 * (_BA_H - 1.0) + 0.5).astype(np.float32)


def _proposal_kernel(s_ref, dx_ref, dy_ref, dw_ref, dh_ref, info_ref, out_ref):
    f32 = jnp.float32
    i32 = jnp.int32
    neg_inf = f32(-jnp.inf)

    im_h = info_ref[0, 0]
    im_w = info_ref[0, 1]
    min_sz = MIN_SIZE * info_ref[0, 2]

    shape = (_ROWS, _LANES)
    row_i = lax.broadcasted_iota(i32, shape, 0)
    lane_i = lax.broadcasted_iota(i32, shape, 1)
    idx = row_i * _LANES + lane_i

    a = idx % _A
    pos = idx // _A
    wi = pos % _W
    hi = pos // _W

    wa = jnp.full(shape, _BA_W[0], f32)
    ha = jnp.full(shape, _BA_H[0], f32)
    cx = jnp.full(shape, _BA_CX[0], f32)
    cy = jnp.full(shape, _BA_CY[0], f32)
    for k in range(1, _A):
        m = a == k
        wa = jnp.where(m, f32(_BA_W[k]), wa)
        ha = jnp.where(m, f32(_BA_H[k]), ha)
        cx = jnp.where(m, f32(_BA_CX[k]), cx)
        cy = jnp.where(m, f32(_BA_CY[k]), cy)
    cx = cx + FEAT_STRIDE * wi.astype(f32)
    cy = cy + FEAT_STRIDE * hi.astype(f32)

    pcx = dx_ref[...] * wa + cx
    pcy = dy_ref[...] * ha + cy
    pw = jnp.exp(dw_ref[...]) * wa
    ph = jnp.exp(dh_ref[...]) * ha
    x1 = jnp.clip(pcx - 0.5 * pw, 0.0, im_w - 1.0)
    y1 = jnp.clip(pcy - 0.5 * ph, 0.0, im_h - 1.0)
    x2 = jnp.clip(pcx + 0.5 * pw, 0.0, im_w - 1.0)
    y2 = jnp.clip(pcy + 0.5 * ph, 0.0, im_h - 1.0)

    ws = x2 - x1 + 1.0
    hs = y2 - y1 + 1.0
    areas = ws * hs

    valid_lane = idx < _NV
    keep = (ws >= min_sz) & (hs >= min_sz) & valid_lane
    s0 = jnp.where(keep, s_ref[...], neg_inf)

    kbits = lax.bitcast_convert_type(s0, i32)
    kbits = jnp.where(kbits < 0, kbits ^ i32(0x7FFFFFFF), kbits)
    min_i32 = i32(-2147483648)
    kbits = jnp.where(valid_lane, kbits, min_i32)

    def _bit_step(b, thr):
        inc = (i32(1) << b).astype(i32)
        t_try = thr + inc
        cnt = jnp.sum((kbits >= t_try).astype(i32))
        return jnp.where(cnt >= PRE_NMS_TOP_N, t_try, thr)

    thr = min_i32
    for b in range(31, -1, -1):
        thr = _bit_step(jnp.int32(b), thr)

    s_init = jnp.where(kbits >= thr, s0, neg_inf)

    big = i32(_NPAD)
    lane_row = lax.broadcasted_iota(i32, (1, _LANES), 1)

    def step(i, s):
        sj = jnp.max(s)
        cand = jnp.where(s == sj, idx, big)
        j = jnp.min(cand)
        sel = (idx == j).astype(f32)
        x1j = jnp.sum(x1 * sel)
        y1j = jnp.sum(y1 * sel)
        x2j = jnp.sum(x2 * sel)
        y2j = jnp.sum(y2 * sel)
        aj = (x2j - x1j + 1.0) * (y2j - y1j + 1.0)

        iw = jnp.maximum(0.0, jnp.minimum(x2j, x2) - jnp.maximum(x1j, x1) + 1.0)
        ih = jnp.maximum(0.0, jnp.minimum(y2j, y2) - jnp.maximum(y1j, y1) + 1.0)
        inter = iw * ih
        iou = inter / (aj + areas - inter)
        s = jnp.where(iou > NMS_THRESH, neg_inf, s)
        s = jnp.where(idx == j, neg_inf, s)

        valid = (sj > neg_inf).astype(f32)
        row = valid * (jnp.where(lane_row == 1, x1j, 0.0)
                       + jnp.where(lane_row == 2, y1j, 0.0)
                       + jnp.where(lane_row == 3, x2j, 0.0)
                       + jnp.where(lane_row == 4, y2j, 0.0))
        out_ref[pl.ds(i, 1), :] = row
        return s

    lax.fori_loop(0, POST_NMS_TOP_N, step, s_init)


@functools.partial(jax.jit, static_argnames=())
def _run(s_flat, dx, dy, dw, dh, im_info):
    pad = _NPAD - _NV

    def to2d(v):
        return jnp.pad(v, (0, pad)).reshape(_ROWS, _LANES)

    out = pl.pallas_call(
        _proposal_kernel,
        out_shape=jax.ShapeDtypeStruct((POST_NMS_TOP_N, _LANES), jnp.float32),
    )(to2d(s_flat), to2d(dx), to2d(dy), to2d(dw), to2d(dh), im_info)
    return out[None, :, :5]


def kernel(scores, bbox_deltas, im_info):
    fg = scores[0, _A:, :, :]
    s_flat = fg.transpose(1, 2, 0).reshape(-1)
    d = bbox_deltas[0].transpose(1, 2, 0).reshape(-1, 4)
    return _run(s_flat, d[:, 0], d[:, 1], d[:, 2], d[:, 3], im_info)
